# baseline (speedup 1.0000x reference)
import functools

import jax
import jax.numpy as jnp
from jax import lax
from jax.experimental import pallas as pl
from jax.experimental.pallas import tpu as pltpu

_EPS = 1e-8
_TK = 128
_ROW_TILE = 512


def _prefix_mat(tk: int) -> jax.Array:
    k = jnp.arange(tk, dtype=jnp.int32)[:, None]
    i = jnp.arange(tk, dtype=jnp.int32)[None, :]
    return (k <= i).astype(jnp.float32)


def _trapz_total(pdf):
    return jnp.sum(pdf, axis=1, keepdims=True) - 0.5 * (pdf[:, :1] + pdf[:, -1:])


def _w1_kernel(d_ref, r_ref, l_ref, out_ref, *, eps, n, tk):
    pd = jnp.abs(d_ref[...].astype(jnp.float32))
    pr = jnp.abs(r_ref[...].astype(jnp.float32))

    inv_d = pl.reciprocal(eps + _trapz_total(pd), approx=False)
    inv_r = pl.reciprocal(eps + _trapz_total(pr), approx=False)

    col = lax.broadcasted_iota(jnp.int32, pd.shape, 1)
    td = jnp.where(col == 0, 0.0, 0.5 * (pltpu.roll(pd, shift=1, axis=1) + pd))
    tr = jnp.where(col == 0, 0.0, 0.5 * (pltpu.roll(pr, shift=1, axis=1) + pr))

    l_mat = l_ref[...]
    rows = pd.shape[0]
    acc = jnp.zeros((rows, 1), jnp.float32)
    carry_d = jnp.zeros((rows, 1), jnp.float32)
    carry_r = jnp.zeros((rows, 1), jnp.float32)
    for c in range(n // tk):
        sl = slice(c * tk, (c + 1) * tk)
        vd = carry_d + jnp.dot(td[:, sl], l_mat, preferred_element_type=jnp.float32)
        vr = carry_r + jnp.dot(tr[:, sl], l_mat, preferred_element_type=jnp.float32)
        diff = vd * inv_d - vr * inv_r
        acc += jnp.sum(diff * diff, axis=1, keepdims=True)
        carry_d = vd[:, -1:]
        carry_r = vr[:, -1:]

    out_ref[...] = acc * (1.0 / (n - 1))


def kernel(data, ref_data):
    B, C, N = data.shape
    rows = B * C
    d = data.reshape(rows, N)
    r = ref_data.reshape(rows, N)
    tile = min(_ROW_TILE, rows)
    l_mat = _prefix_mat(_TK)

    per_row = pl.pallas_call(
        functools.partial(_w1_kernel, eps=_EPS, n=N, tk=_TK),
        out_shape=jax.ShapeDtypeStruct((rows, 1), jnp.float32),
        grid=(pl.cdiv(rows, tile),),
        in_specs=[
            pl.BlockSpec((tile, N), lambda i: (i, 0)),
            pl.BlockSpec((tile, N), lambda i: (i, 0)),
            pl.BlockSpec((_TK, _TK), lambda i: (0, 0), pipeline_mode=pl.Buffered(1)),
        ],
        out_specs=pl.BlockSpec((tile, 1), lambda i: (i, 0)),
        compiler_params=pltpu.CompilerParams(
            dimension_semantics=("parallel",),
            vmem_limit_bytes=48 * 1024 * 1024),
        cost_estimate=pl.CostEstimate(
            flops=2 * 2 * rows * N * _TK + 12 * rows * N,
            transcendentals=0,
            bytes_accessed=(d.size + r.size) * d.dtype.itemsize + 4 * rows),
    )(d, r, l_mat)

    return per_row[:, 0].reshape(B, C).mean(axis=1)


# speedup vs baseline: 1.8193x; 1.8193x over previous
import functools

import jax
import jax.numpy as jnp
from jax.experimental import pallas as pl
from jax.experimental.pallas import tpu as pltpu

_EPS = 1e-8
_ROW_TILE = 1024


def _make_w(n: int) -> jax.Array:
    nm1 = n - 1
    k = jnp.arange(n, dtype=jnp.int32)[:, None]
    i = jnp.arange(n, dtype=jnp.int32)[None, :]
    w = jnp.where(k <= i, 1.0, 0.0)
    w = jnp.where((k == 0) | (k == i + 1), 0.5, w)
    w = jnp.where(i >= nm1, 0.0, w)
    return w.astype(jnp.bfloat16)


def _trapz_total(pdf):
    return jnp.sum(pdf, axis=1, keepdims=True) - 0.5 * (pdf[:, :1] + pdf[:, -1:])


def _w1_kernel(d_ref, r_ref, w_ref, out_ref, *, eps, n):
    ad = jnp.abs(d_ref[...].astype(jnp.float32))
    ar = jnp.abs(r_ref[...].astype(jnp.float32))

    inv_d = pl.reciprocal(eps + _trapz_total(ad), approx=False)
    inv_r = pl.reciprocal(eps + _trapz_total(ar), approx=False)

    w = w_ref[...]
    vd = jnp.dot(ad.astype(jnp.bfloat16), w, preferred_element_type=jnp.float32)
    vr = jnp.dot(ar.astype(jnp.bfloat16), w, preferred_element_type=jnp.float32)

    diff = vd * inv_d - vr * inv_r
    out_ref[...] = jnp.sum(diff * diff, axis=1, keepdims=True) * (1.0 / (n - 1))


def kernel(data, ref_data):
    B, C, N = data.shape
    rows = B * C
    d = data.reshape(rows, N)
    r = ref_data.reshape(rows, N)
    tile = min(_ROW_TILE, rows)
    w = _make_w(N)

    per_row = pl.pallas_call(
        functools.partial(_w1_kernel, eps=_EPS, n=N),
        out_shape=jax.ShapeDtypeStruct((rows, 1), jnp.float32),
        grid=(pl.cdiv(rows, tile),),
        in_specs=[
            pl.BlockSpec((tile, N), lambda i: (i, 0)),
            pl.BlockSpec((tile, N), lambda i: (i, 0)),
            pl.BlockSpec((N, N), lambda i: (0, 0), pipeline_mode=pl.Buffered(1)),
        ],
        out_specs=pl.BlockSpec((tile, 1), lambda i: (i, 0)),
        compiler_params=pltpu.CompilerParams(
            dimension_semantics=("parallel",),
            vmem_limit_bytes=48 * 1024 * 1024),
        cost_estimate=pl.CostEstimate(
            flops=2 * 2 * rows * N * N + 12 * rows * N,
            transcendentals=0,
            bytes_accessed=(d.size + r.size) * d.dtype.itemsize + 4 * rows),
    )(d, r, w)

    return per_row[:, 0].reshape(B, C).mean(axis=1)


# speedup vs baseline: 1.8572x; 1.0208x over previous
import functools

import jax
import jax.numpy as jnp
from jax.experimental import pallas as pl
from jax.experimental.pallas import tpu as pltpu

_EPS = 1e-8
_ROW_TILE = 1024


def _make_w(n: int) -> jax.Array:
    nm1 = n - 1
    k = jnp.arange(n, dtype=jnp.int32)[:, None]
    i = jnp.arange(n, dtype=jnp.int32)[None, :]
    w = jnp.where(k <= i, 1.0, 0.0)
    w = jnp.where((k == 0) | (k == i + 1), 0.5, w)
    w = jnp.where(i >= nm1, 0.0, w)
    return w.astype(jnp.bfloat16)


def _trapz_total(pdf):
    return jnp.sum(pdf, axis=1, keepdims=True) - 0.5 * (pdf[:, :1] + pdf[:, -1:])


def _w1_kernel(d_ref, r_ref, w_ref, out_ref, *, eps, n):
    ad = jnp.abs(d_ref[...].astype(jnp.float32))
    ar = jnp.abs(r_ref[...].astype(jnp.float32))

    inv_d = pl.reciprocal(eps + _trapz_total(ad), approx=False)
    inv_r = pl.reciprocal(eps + _trapz_total(ar), approx=False)

    s = (ad * inv_d - ar * inv_r).astype(jnp.bfloat16)
    diff = jnp.dot(s, w_ref[...], preferred_element_type=jnp.float32)
    out_ref[...] = jnp.sum(diff * diff, axis=1, keepdims=True) * (1.0 / (n - 1))


def kernel(data, ref_data):
    B, C, N = data.shape
    rows = B * C
    d = data.reshape(rows, N)
    r = ref_data.reshape(rows, N)
    tile = min(_ROW_TILE, rows)
    w = _make_w(N)

    per_row = pl.pallas_call(
        functools.partial(_w1_kernel, eps=_EPS, n=N),
        out_shape=jax.ShapeDtypeStruct((rows, 1), jnp.float32),
        grid=(pl.cdiv(rows, tile),),
        in_specs=[
            pl.BlockSpec((tile, N), lambda i: (i, 0)),
            pl.BlockSpec((tile, N), lambda i: (i, 0)),
            pl.BlockSpec((N, N), lambda i: (0, 0), pipeline_mode=pl.Buffered(1)),
        ],
        out_specs=pl.BlockSpec((tile, 1), lambda i: (i, 0)),
        compiler_params=pltpu.CompilerParams(
            dimension_semantics=("parallel",),
            vmem_limit_bytes=48 * 1024 * 1024),
        cost_estimate=pl.CostEstimate(
            flops=2 * 2 * rows * N * N + 12 * rows * N,
            transcendentals=0,
            bytes_accessed=(d.size + r.size) * d.dtype.itemsize + 4 * rows),
    )(d, r, w)

    return per_row[:, 0].reshape(B, C).mean(axis=1)
